# baseline (speedup 1.0000x reference)
import functools

import jax
import jax.numpy as jnp
from jax.experimental import pallas as pl

_REG = 17
_C = 80
_K = 300
_IOU_THR = 0.7
_NEG = float("-inf")


def _reduce_kernel(cls_ref, out_ref):
    out_ref[0] = jnp.max(cls_ref[...], axis=-1)


def _topk_kernel(m_ref, vals_ref, idx_ref):
    b, l = m_ref.shape
    lane = jax.lax.broadcasted_iota(jnp.int32, (b, l), 1)
    klane = jax.lax.broadcasted_iota(jnp.int32, (b, _K), 1)

    def body(k, carry):
        x, vals, idx = carry
        v = jnp.max(x, axis=1, keepdims=True)
        cand = jnp.where(x == v, lane, l)
        j = jnp.min(cand, axis=1, keepdims=True)
        vals = jnp.where(klane == k, v, vals)
        idx = jnp.where(klane == k, j, idx)
        x = jnp.where(lane == j, _NEG, x)
        return x, vals, idx

    x0 = m_ref[...]
    vals0 = jnp.zeros((b, _K), jnp.float32)
    idx0 = jnp.zeros((b, _K), jnp.int32)
    _, vals, idx = jax.lax.fori_loop(0, _K, body, (x0, vals0, idx0))
    vals_ref[...] = 1.0 / (1.0 + jnp.exp(-vals))
    idx_ref[...] = idx


def _decode_nms_kernel(pred_ref, idx_ref, vals_ref, anch_ref, str_ref,
                       proj_ref, out_ref):
    l = pred_ref.shape[1]
    idx_col = jnp.reshape(idx_ref[...], (_K, 1))

    nchunk = 4
    ch = l // nchunk
    acc = jnp.zeros((_K, 4 * _REG), jnp.float32)
    acc_a = jnp.zeros((_K, 2), jnp.float32)
    acc_s = jnp.zeros((_K, 1), jnp.float32)
    for c in range(nchunk):
        seg = jax.lax.broadcasted_iota(jnp.int32, (_K, ch), 1) + c * ch
        oh = (seg == idx_col).astype(jnp.float32)
        acc = acc + jnp.dot(oh, pred_ref[0, c * ch:(c + 1) * ch, :],
                            preferred_element_type=jnp.float32)
        acc_a = acc_a + jnp.dot(oh, anch_ref[c * ch:(c + 1) * ch, :],
                                preferred_element_type=jnp.float32)
        acc_s = acc_s + jnp.dot(oh, str_ref[c * ch:(c + 1) * ch, :],
                                preferred_element_type=jnp.float32)

    projrow = proj_ref[...]
    dists = []
    for s in range(4):
        d = acc[:, s * _REG:(s + 1) * _REG]
        d = d - jnp.max(d, axis=1, keepdims=True)
        e = jnp.exp(d)
        dists.append(jnp.sum(e * projrow, axis=1, keepdims=True)
                     / jnp.sum(e, axis=1, keepdims=True))

    ax = acc_a[:, 0:1]
    ay = acc_a[:, 1:2]
    x1 = (ax - dists[0]) * acc_s
    y1 = (ay - dists[1]) * acc_s
    x2 = (ax + dists[2]) * acc_s
    y2 = (ay + dists[3]) * acc_s

    x1r = jnp.reshape(x1, (1, _K))
    y1r = jnp.reshape(y1, (1, _K))
    x2r = jnp.reshape(x2, (1, _K))
    y2r = jnp.reshape(y2, (1, _K))
    w = jnp.clip(jnp.minimum(x2, x2r) - jnp.maximum(x1, x1r), 0.0, None)
    h = jnp.clip(jnp.minimum(y2, y2r) - jnp.maximum(y1, y1r), 0.0, None)
    inter = w * h
    area = (x2 - x1) * (y2 - y1)
    union = area + jnp.reshape(area, (1, _K)) - inter + 1e-10
    iou = inter / union
    rr = jax.lax.broadcasted_iota(jnp.int32, (_K, _K), 0)
    cc = jax.lax.broadcasted_iota(jnp.int32, (_K, _K), 1)
    iou = jnp.where(rr < cc, iou, 0.0)
    keep = (jnp.max(iou, axis=0, keepdims=True) <= _IOU_THR)
    final = vals_ref[...].reshape(1, _K) * keep.astype(jnp.float32)
    out_ref[0] = jnp.concatenate(
        [x1, y1, x2, y2, jnp.reshape(final, (_K, 1))], axis=1)


@jax.jit
def kernel(pred_dist, cls_logits, anchor_points, stride_tensor, proj):
    b, l, _ = cls_logits.shape
    ch = 400
    nch = l // ch
    m3 = pl.pallas_call(
        _reduce_kernel,
        grid=(nch,),
        in_specs=[pl.BlockSpec((b, ch, _C), lambda i: (0, i, 0))],
        out_specs=pl.BlockSpec((1, b, ch), lambda i: (i, 0, 0)),
        out_shape=jax.ShapeDtypeStruct((nch, b, ch), jnp.float32),
    )(cls_logits)
    m = m3.transpose(1, 0, 2).reshape(b, l)

    vals, idx = pl.pallas_call(
        _topk_kernel,
        out_shape=(jax.ShapeDtypeStruct((b, _K), jnp.float32),
                   jax.ShapeDtypeStruct((b, _K), jnp.int32)),
    )(m)

    out = pl.pallas_call(
        _decode_nms_kernel,
        grid=(b,),
        in_specs=[
            pl.BlockSpec((1, l, 4 * _REG), lambda i: (i, 0, 0)),
            pl.BlockSpec((1, 1, _K), lambda i: (i, 0, 0)),
            pl.BlockSpec((1, 1, _K), lambda i: (i, 0, 0)),
            pl.BlockSpec((l, 2), lambda i: (0, 0)),
            pl.BlockSpec((l, 1), lambda i: (0, 0)),
            pl.BlockSpec((1, _REG), lambda i: (0, 0)),
        ],
        out_specs=pl.BlockSpec((1, _K, 5), lambda i: (i, 0, 0)),
        out_shape=jax.ShapeDtypeStruct((b, _K, 5), jnp.float32),
    )(pred_dist, idx.reshape(b, 1, _K), vals.reshape(b, 1, _K),
      anchor_points, stride_tensor, proj.reshape(1, _REG))
    return out


# speedup vs baseline: 1.0062x; 1.0062x over previous
import functools

import jax
import jax.numpy as jnp
from jax.experimental import pallas as pl

_REG = 17
_C = 80
_K = 300
_IOU_THR = 0.7
_NEG = float("-inf")


def _reduce_kernel(cls_ref, out_ref):
    out_ref[0] = jnp.max(cls_ref[...], axis=-1)


def _topk_kernel(m_ref, vals_ref, idx_ref):
    b, l = m_ref.shape
    klane = jax.lax.broadcasted_iota(jnp.int32, (b, _K), 1)

    def body(k, carry):
        x, v, vals, idx = carry
        lane = jax.lax.broadcasted_iota(jnp.int32, (b, l), 1)
        j = jnp.min(jnp.where(x == v, lane, l), axis=1, keepdims=True)
        x = jnp.where(lane == j, _NEG, x)
        vn = jnp.max(x, axis=1, keepdims=True)
        vals = jnp.where(klane == k, v, vals)
        idx = jnp.where(klane == k, j, idx)
        return x, vn, vals, idx

    x0 = m_ref[...]
    v0 = jnp.max(x0, axis=1, keepdims=True)
    vals0 = jnp.zeros((b, _K), jnp.float32)
    idx0 = jnp.zeros((b, _K), jnp.int32)
    _, _, vals, idx = jax.lax.fori_loop(0, _K, body,
                                        (x0, v0, vals0, idx0))
    vals_ref[...] = 1.0 / (1.0 + jnp.exp(-vals))
    idx_ref[...] = idx


def _decode_nms_kernel(pred_ref, idx_ref, vals_ref, anch_ref, str_ref,
                       proj_ref, out_ref):
    l = pred_ref.shape[1]
    idx_col = jnp.reshape(idx_ref[...], (_K, 1))

    nchunk = 4
    ch = l // nchunk
    acc = jnp.zeros((_K, 4 * _REG), jnp.float32)
    acc_a = jnp.zeros((_K, 2), jnp.float32)
    acc_s = jnp.zeros((_K, 1), jnp.float32)
    for c in range(nchunk):
        seg = jax.lax.broadcasted_iota(jnp.int32, (_K, ch), 1) + c * ch
        oh = (seg == idx_col).astype(jnp.float32)
        acc = acc + jnp.dot(oh, pred_ref[0, c * ch:(c + 1) * ch, :],
                            preferred_element_type=jnp.float32)
        acc_a = acc_a + jnp.dot(oh, anch_ref[c * ch:(c + 1) * ch, :],
                                preferred_element_type=jnp.float32)
        acc_s = acc_s + jnp.dot(oh, str_ref[c * ch:(c + 1) * ch, :],
                                preferred_element_type=jnp.float32)

    projrow = proj_ref[...]
    dists = []
    for s in range(4):
        d = acc[:, s * _REG:(s + 1) * _REG]
        d = d - jnp.max(d, axis=1, keepdims=True)
        e = jnp.exp(d)
        dists.append(jnp.sum(e * projrow, axis=1, keepdims=True)
                     / jnp.sum(e, axis=1, keepdims=True))

    ax = acc_a[:, 0:1]
    ay = acc_a[:, 1:2]
    x1 = (ax - dists[0]) * acc_s
    y1 = (ay - dists[1]) * acc_s
    x2 = (ax + dists[2]) * acc_s
    y2 = (ay + dists[3]) * acc_s

    x1r = jnp.reshape(x1, (1, _K))
    y1r = jnp.reshape(y1, (1, _K))
    x2r = jnp.reshape(x2, (1, _K))
    y2r = jnp.reshape(y2, (1, _K))
    w = jnp.clip(jnp.minimum(x2, x2r) - jnp.maximum(x1, x1r), 0.0, None)
    h = jnp.clip(jnp.minimum(y2, y2r) - jnp.maximum(y1, y1r), 0.0, None)
    inter = w * h
    area = (x2 - x1) * (y2 - y1)
    union = area + jnp.reshape(area, (1, _K)) - inter + 1e-10
    iou = inter / union
    rr = jax.lax.broadcasted_iota(jnp.int32, (_K, _K), 0)
    cc = jax.lax.broadcasted_iota(jnp.int32, (_K, _K), 1)
    iou = jnp.where(rr < cc, iou, 0.0)
    keep = (jnp.max(iou, axis=0, keepdims=True) <= _IOU_THR)
    final = vals_ref[...].reshape(1, _K) * keep.astype(jnp.float32)
    out_ref[0] = jnp.concatenate(
        [x1, y1, x2, y2, jnp.reshape(final, (_K, 1))], axis=1)


@jax.jit
def kernel(pred_dist, cls_logits, anchor_points, stride_tensor, proj):
    b, l, _ = cls_logits.shape
    ch = 400
    nch = l // ch
    m3 = pl.pallas_call(
        _reduce_kernel,
        grid=(nch,),
        in_specs=[pl.BlockSpec((b, ch, _C), lambda i: (0, i, 0))],
        out_specs=pl.BlockSpec((1, b, ch), lambda i: (i, 0, 0)),
        out_shape=jax.ShapeDtypeStruct((nch, b, ch), jnp.float32),
    )(cls_logits)
    m = m3.transpose(1, 0, 2).reshape(b, l)

    vals, idx = pl.pallas_call(
        _topk_kernel,
        out_shape=(jax.ShapeDtypeStruct((b, _K), jnp.float32),
                   jax.ShapeDtypeStruct((b, _K), jnp.int32)),
    )(m)

    out = pl.pallas_call(
        _decode_nms_kernel,
        grid=(b,),
        in_specs=[
            pl.BlockSpec((1, l, 4 * _REG), lambda i: (i, 0, 0)),
            pl.BlockSpec((1, 1, _K), lambda i: (i, 0, 0)),
            pl.BlockSpec((1, 1, _K), lambda i: (i, 0, 0)),
            pl.BlockSpec((l, 2), lambda i: (0, 0)),
            pl.BlockSpec((l, 1), lambda i: (0, 0)),
            pl.BlockSpec((1, _REG), lambda i: (0, 0)),
        ],
        out_specs=pl.BlockSpec((1, _K, 5), lambda i: (i, 0, 0)),
        out_shape=jax.ShapeDtypeStruct((b, _K, 5), jnp.float32),
    )(pred_dist, idx.reshape(b, 1, _K), vals.reshape(b, 1, _K),
      anchor_points, stride_tensor, proj.reshape(1, _REG))
    return out
